# Initial kernel scaffold; baseline (speedup 1.0000x reference)
import jax
import jax.numpy as jnp
from jax.experimental import pallas as pl


def kernel(input_ids, token_embeddings, position_embeddings):
    raise NotImplementedError("write your pallas kernel here")


# speedup vs baseline: 1.0721x; 1.0721x over previous
import functools

import jax
import jax.numpy as jnp
from jax import lax
from jax.experimental import pallas as pl
from jax.experimental.pallas import tpu as pltpu
from jax.experimental.pallas import tpu_sc as plsc

D = 128
B = 4
S = 2048
TOTAL = B * S
NC = 2
NS = 16
L = 16
NW = NC * NS
BPW = TOTAL // NW
CHUNK = 128
NCHUNK = BPW // CHUNK


def _build():
    mesh = plsc.VectorSubcoreMesh(core_axis_name="c", subcore_axis_name="s")

    @functools.partial(
        pl.kernel,
        mesh=mesh,
        out_type=jax.ShapeDtypeStruct((TOTAL, D), jnp.float32),
        scratch_types=[
            pltpu.VMEM((NCHUNK, CHUNK), jnp.int32),
            pltpu.VMEM((BPW, D), jnp.float32),
            pltpu.VMEM((BPW, D), jnp.float32),
            pltpu.SemaphoreType.DMA,
        ],
    )
    def emb_kernel(idx_hbm, table_hbm, pos_hbm, out_hbm, idx_v, rows_v, pos_v, sem):
        c = lax.axis_index("c")
        s = lax.axis_index("s")
        wid = c * NS + s
        gbase = wid * BPW
        pbase = lax.rem(gbase, S)

        pltpu.sync_copy(idx_hbm.at[pl.ds(wid * NCHUNK, NCHUNK)], idx_v)
        ppos = pltpu.async_copy(pos_hbm.at[pl.ds(pbase, BPW)], pos_v, sem)
        gathers = [
            pltpu.async_copy(
                table_hbm.at[idx_v.at[j]],
                rows_v.at[pl.ds(j * CHUNK, CHUNK)],
                sem,
            )
            for j in range(NCHUNK)
        ]
        ppos.wait()
        for g in gathers:
            g.wait()

        def add_row(r, carry):
            for cc in range(D // L):
                sl = pl.ds(cc * L, L)
                rows_v[r, sl] = rows_v[r, sl] + pos_v[r, sl]
            return carry

        lax.fori_loop(0, BPW, add_row, 0)

        pltpu.sync_copy(rows_v, out_hbm.at[pl.ds(gbase, BPW)])

    return emb_kernel


_emb_kernel = _build()


@jax.jit
def kernel(input_ids, token_embeddings, position_embeddings):
    idx = input_ids.reshape(NW * NCHUNK, CHUNK).astype(jnp.int32)
    out = _emb_kernel(idx, token_embeddings, position_embeddings)
    return out.reshape(B, S, D)


# speedup vs baseline: 1.0974x; 1.0236x over previous
import functools

import jax
import jax.numpy as jnp
from jax import lax
from jax.experimental import pallas as pl
from jax.experimental.pallas import tpu as pltpu
from jax.experimental.pallas import tpu_sc as plsc

D = 128
B = 4
S = 2048
TOTAL = B * S
NC = 2
NS = 16
L = 16
NW = NC * NS
BPW = TOTAL // NW
CHUNK = 64
NCHUNK = BPW // CHUNK


def _build():
    mesh = plsc.VectorSubcoreMesh(core_axis_name="c", subcore_axis_name="s")

    @functools.partial(
        pl.kernel,
        mesh=mesh,
        out_type=jax.ShapeDtypeStruct((B, S, D), jnp.float32),
        scratch_types=[
            pltpu.VMEM((BPW,), jnp.int32),
            pltpu.VMEM((BPW, D), jnp.float32),
            pltpu.VMEM((BPW, D), jnp.float32),
            [pltpu.SemaphoreType.DMA] * NCHUNK,
            pltpu.SemaphoreType.DMA,
            pltpu.SemaphoreType.DMA,
        ],
    )
    def emb_kernel(idx_hbm, table_hbm, pos_hbm, out_hbm,
                   idx_v, rows_v, pos_v, gsems, psem, osem):
        c = lax.axis_index("c")
        s = lax.axis_index("s")
        wid = c * NS + s
        gbase = wid * BPW
        b = lax.div(gbase, S)
        off = lax.rem(gbase, S)

        pltpu.sync_copy(idx_hbm.at[b, pl.ds(off, BPW)], idx_v)
        pos_cp = pltpu.async_copy(pos_hbm.at[pl.ds(off, BPW)], pos_v, psem)
        gathers = [
            pltpu.async_copy(
                table_hbm.at[idx_v.at[pl.ds(j * CHUNK, CHUNK)]],
                rows_v.at[pl.ds(j * CHUNK, CHUNK)],
                gsems[j],
            )
            for j in range(NCHUNK)
        ]
        pos_cp.wait()

        outs = []
        for j in range(NCHUNK):
            gathers[j].wait()
            base = j * CHUNK

            def add_row(r, carry):
                for cc in range(D // L):
                    sl = pl.ds(cc * L, L)
                    rows_v[r, sl] = rows_v[r, sl] + pos_v[r, sl]
                return carry

            lax.fori_loop(base, base + CHUNK, add_row, 0)
            outs.append(pltpu.async_copy(
                rows_v.at[pl.ds(base, CHUNK)],
                out_hbm.at[b, pl.ds(off + base, CHUNK)],
                osem,
            ))
        for o in outs:
            o.wait()

    return emb_kernel


_emb_kernel = _build()


def kernel(input_ids, token_embeddings, position_embeddings):
    return _emb_kernel(input_ids.astype(jnp.int32), token_embeddings,
                       position_embeddings)


# speedup vs baseline: 1.1419x; 1.0406x over previous
import functools

import jax
import jax.numpy as jnp
from jax import lax
from jax.experimental import pallas as pl
from jax.experimental.pallas import tpu as pltpu
from jax.experimental.pallas import tpu_sc as plsc

D = 128
B = 4
S = 2048
TOTAL = B * S
NC = 2
NS = 16
L = 16
NW = NC * NS
BPW = TOTAL // NW
CHUNK = 64
NCHUNK = BPW // CHUNK


def _build():
    mesh = plsc.VectorSubcoreMesh(core_axis_name="c", subcore_axis_name="s")

    @functools.partial(
        pl.kernel,
        mesh=mesh,
        out_type=jax.ShapeDtypeStruct((B, S, D), jnp.float32),
        scratch_types=[
            pltpu.VMEM((BPW,), jnp.int32),
            pltpu.VMEM((BPW, D), jnp.float32),
            pltpu.VMEM((BPW, D), jnp.float32),
            [pltpu.SemaphoreType.DMA] * NCHUNK,
            pltpu.SemaphoreType.DMA,
            pltpu.SemaphoreType.DMA,
        ],
    )
    def emb_kernel(idx_hbm, table_hbm, pos_hbm, out_hbm,
                   idx_v, rows_v, pos_v, gsems, psem, osem):
        c = lax.axis_index("c")
        s = lax.axis_index("s")
        wid = c * NS + s
        gbase = wid * BPW
        b = lax.div(gbase, S)
        off = lax.rem(gbase, S)

        pltpu.sync_copy(idx_hbm.at[b, pl.ds(off, BPW)], idx_v)
        pos_cp = pltpu.async_copy(pos_hbm.at[pl.ds(off, BPW)], rows_v, psem)
        pos_cp.wait()
        gathers = [
            pltpu.async_copy(
                table_hbm.at[idx_v.at[pl.ds(j * CHUNK, CHUNK)]],
                rows_v.at[pl.ds(j * CHUNK, CHUNK)],
                gsems[j],
                add=True,
            )
            for j in range(NCHUNK)
        ]
        outs = []
        for j in range(NCHUNK):
            gathers[j].wait()
            base = j * CHUNK
            outs.append(pltpu.async_copy(
                rows_v.at[pl.ds(base, CHUNK)],
                out_hbm.at[b, pl.ds(off + base, CHUNK)],
                osem,
            ))
        for o in outs:
            o.wait()

    return emb_kernel


_emb_kernel = _build()


def kernel(input_ids, token_embeddings, position_embeddings):
    return _emb_kernel(input_ids.astype(jnp.int32), token_embeddings,
                       position_embeddings)


# speedup vs baseline: 1.1748x; 1.0288x over previous
import functools

import jax
import jax.numpy as jnp
from jax import lax
from jax.experimental import pallas as pl
from jax.experimental.pallas import tpu as pltpu
from jax.experimental.pallas import tpu_sc as plsc

D = 128
B = 4
S = 2048
TOTAL = B * S
NC = 2
NS = 16
L = 16
NW = NC * NS
BPW = TOTAL // NW
CHUNK = 64
NCHUNK = BPW // CHUNK


def _build():
    mesh = plsc.VectorSubcoreMesh(core_axis_name="c", subcore_axis_name="s")

    @functools.partial(
        pl.kernel,
        mesh=mesh,
        out_type=jax.ShapeDtypeStruct((B, S, D), jnp.float32),
        scratch_types=[
            pltpu.VMEM((BPW,), jnp.int32),
            pltpu.VMEM((BPW, D), jnp.float32),
            [pltpu.SemaphoreType.DMA] * NCHUNK,
            [pltpu.SemaphoreType.DMA] * NCHUNK,
            pltpu.SemaphoreType.DMA,
            pltpu.SemaphoreType.DMA,
        ],
    )
    def emb_kernel(idx_hbm, table_hbm, pos_hbm, out_hbm,
                   idx_v, rows_v, gsems, psems, isem, osem):
        c = lax.axis_index("c")
        s = lax.axis_index("s")
        wid = c * NS + s
        gbase = wid * BPW
        b = lax.div(gbase, S)
        off = lax.rem(gbase, S)

        idx_cp = pltpu.async_copy(idx_hbm.at[b, pl.ds(off, BPW)], idx_v, isem)
        pos_cps = [
            pltpu.async_copy(
                pos_hbm.at[pl.ds(off + j * CHUNK, CHUNK)],
                rows_v.at[pl.ds(j * CHUNK, CHUNK)],
                psems[j],
            )
            for j in range(NCHUNK)
        ]
        idx_cp.wait()

        gathers = []
        for j in range(NCHUNK):
            pos_cps[j].wait()
            gathers.append(pltpu.async_copy(
                table_hbm.at[idx_v.at[pl.ds(j * CHUNK, CHUNK)]],
                rows_v.at[pl.ds(j * CHUNK, CHUNK)],
                gsems[j],
                add=True,
            ))
        outs = []
        for j in range(NCHUNK):
            gathers[j].wait()
            base = j * CHUNK
            outs.append(pltpu.async_copy(
                rows_v.at[pl.ds(base, CHUNK)],
                out_hbm.at[b, pl.ds(off + base, CHUNK)],
                osem,
            ))
        for o in outs:
            o.wait()

    return emb_kernel


_emb_kernel = _build()


def kernel(input_ids, token_embeddings, position_embeddings):
    return _emb_kernel(input_ids.astype(jnp.int32), token_embeddings,
                       position_embeddings)
